# baseline (speedup 1.0000x reference)
import functools

import jax
import jax.numpy as jnp
from jax import lax
from jax.experimental import layout as jex_layout
from jax.experimental import pallas as pl
from jax.experimental.pallas import tpu as pltpu
from jax.experimental.pallas import tpu_sc as plsc

VOCAB = 100000
EMBED = 192
B = 16384
C = 5

NC = 2
NS = 16
NW = NC * NS
BPW = B // NW
CB = 16
NCHUNK = BPW // CB
CV = CB * C
NR = 4
W = EMBED // 2
EW = W // 16


def _w2v_body(tgt_idx_hbm, ctx_idx_hbm, tgt_tab_hbm, ctx_tab_hbm, out_hbm,
              tgt_idx_v, ctx_idx_v, tgt_rows_v, ctx_rows_v, out_v,
              sem_t, sem_c):
    cid = lax.axis_index("c")
    sid = lax.axis_index("s")
    wid = sid * NC + cid
    b0 = wid * BPW

    pltpu.sync_copy(tgt_idx_hbm.at[pl.ds(b0, BPW)], tgt_idx_v)
    pltpu.sync_copy(ctx_idx_hbm.at[pl.ds(b0 * C, BPW * C)], ctx_idx_v)

    def descriptors(g, slot):
        ti = tgt_idx_v.at[pl.ds(g * CB, CB)]
        ci = ctx_idx_v.at[pl.ds(g * CV, CV)]
        return (
            pltpu.make_async_copy(
                tgt_tab_hbm.at[ti], tgt_rows_v.at[slot], sem_t[slot]),
            pltpu.make_async_copy(
                ctx_tab_hbm.at[ci], ctx_rows_v.at[slot], sem_c[slot]),
        )

    def fire(g, slot):
        for cp in descriptors(g, slot):
            cp.start()

    def unpack_row(row_ref, r):
        halves = []
        for e in range(EW):
            words = row_ref[r, pl.ds(e * 16, 16)]
            halves.append(plsc.unpack(plsc.bitcast(words, jnp.bfloat16),
                                      format=plsc.PackFormat.INTERLEAVED))
        return halves

    def compute(g, slot):
        lanes = lax.iota(jnp.int32, 16)
        trows = tgt_rows_v.at[slot]
        crows = ctx_rows_v.at[slot]

        def b_body(i, carry2):
            tvs = unpack_row(trows, i)
            sums = []
            for c in range(C):
                cvs = unpack_row(crows, i * C + c)
                acc = None
                for e in range(EW):
                    part = cvs[e][0] * tvs[e][0] + cvs[e][1] * tvs[e][1]
                    acc = part if acc is None else acc + part
                sums.append(jnp.sum(acc))
            val = jnp.full((16,), sums[0], dtype=jnp.float32)
            for c in range(1, C):
                val = jnp.where(lanes == c, sums[c], val)
            idx = g * CV + i * C + lanes
            plsc.store_scatter(out_v, [idx], val, mask=lanes < C)
            return carry2

        lax.fori_loop(0, CB, b_body, 0, unroll=True)

    for r in range(NR - 1):
        fire(r, r)

    def outer(go, carry):
        for r in range(NR):
            g = go * NR + r
            gp = g + NR - 1

            @pl.when(gp < NCHUNK)
            def _():
                fire(gp, (r + NR - 1) % NR)

            for cp in descriptors(g, r):
                cp.wait()
            compute(g, r)
        return carry

    lax.fori_loop(0, NCHUNK // NR, outer, 0)

    pltpu.sync_copy(out_v, out_hbm.at[pl.ds(b0 * C, BPW * C)])


@functools.cache
def _w2v_call():
    return functools.partial(
        pl.kernel,
        out_type=jax.ShapeDtypeStruct((B * C,), jnp.float32),
        scratch_types=[
            pltpu.VMEM((BPW,), jnp.int32),
            pltpu.VMEM((BPW * C,), jnp.int32),
            pltpu.VMEM((NR, CB, W), jnp.float32),
            pltpu.VMEM((NR, CV, W), jnp.float32),
            pltpu.VMEM((BPW * C,), jnp.float32),
            [pltpu.SemaphoreType.DMA] * NR,
            [pltpu.SemaphoreType.DMA] * NR,
        ],
        mesh=plsc.VectorSubcoreMesh(core_axis_name="c", subcore_axis_name="s"),
        compiler_params=pltpu.CompilerParams(
            needs_layout_passes=False, use_tc_tiling_on_sc=False),
    )(_w2v_body)


def _pack_table(table):
    u = jax.lax.bitcast_convert_type(table.astype(jnp.bfloat16), jnp.uint16)
    words = u[:, 0::2].astype(jnp.uint32) | (
        u[:, 1::2].astype(jnp.uint32) << 16)
    packed = jax.lax.bitcast_convert_type(words, jnp.float32)
    return jex_layout.with_layout_constraint(
        packed, jex_layout.Layout(major_to_minor=(1, 0)))


@jax.jit
def kernel(target, context, target_table, context_table):
    tgt_idx = target.reshape(B).astype(jnp.int32)
    ctx_idx = context.reshape(B * C).astype(jnp.int32)
    out = _w2v_call()(tgt_idx, ctx_idx,
                      _pack_table(target_table), _pack_table(context_table))
    return out.reshape(B, C)


# speedup vs baseline: 2.9799x; 2.9799x over previous
import functools

import jax
import jax.numpy as jnp
from jax import lax
from jax.experimental import layout as jex_layout
from jax.experimental import pallas as pl
from jax.experimental.pallas import tpu as pltpu
from jax.experimental.pallas import tpu_sc as plsc

VOCAB = 100000
EMBED = 192
B = 16384
C = 5

NC = 2
NS = 16
NW = NC * NS
BPW = B // NW
CB = 16
NCHUNK = BPW // CB
CV = CB * C
NR = 4
EA = 8
EB = 4


def _w2v_body(tgt_idx_hbm, ctx_idx_hbm, tgt_a_hbm, tgt_b_hbm,
              ctx_a_hbm, ctx_b_hbm, out_hbm,
              tgt_idx_v, ctx_idx_v, tgt_a_v, tgt_b_v, ctx_a_v, ctx_b_v,
              out_v, sem_ta, sem_tb, sem_ca, sem_cb):
    cid = lax.axis_index("c")
    sid = lax.axis_index("s")
    wid = sid * NC + cid
    b0 = wid * BPW

    pltpu.sync_copy(tgt_idx_hbm.at[pl.ds(b0, BPW)], tgt_idx_v)
    pltpu.sync_copy(ctx_idx_hbm.at[pl.ds(b0 * C, BPW * C)], ctx_idx_v)

    def descriptors(g, slot):
        ti = tgt_idx_v.at[pl.ds(g * CB, CB)]
        ci = ctx_idx_v.at[pl.ds(g * CV, CV)]
        return (
            pltpu.make_async_copy(
                tgt_a_hbm.at[ti], tgt_a_v.at[slot], sem_ta[slot]),
            pltpu.make_async_copy(
                tgt_b_hbm.at[ti], tgt_b_v.at[slot], sem_tb[slot]),
            pltpu.make_async_copy(
                ctx_a_hbm.at[ci], ctx_a_v.at[slot], sem_ca[slot]),
            pltpu.make_async_copy(
                ctx_b_hbm.at[ci], ctx_b_v.at[slot], sem_cb[slot]),
        )

    def fire(g, slot):
        for cp in descriptors(g, slot):
            cp.start()

    def compute(g, slot):
        lanes = lax.iota(jnp.int32, 16)
        t_a = tgt_a_v.at[slot]
        t_b = tgt_b_v.at[slot]
        c_a = ctx_a_v.at[slot]
        c_b = ctx_b_v.at[slot]

        def b_body(i, carry2):
            tvs = [t_a[i, pl.ds(e * 16, 16)] for e in range(EA)]
            tvs += [t_b[i, pl.ds(e * 16, 16)] for e in range(EB)]
            sums = []
            for c in range(C):
                acc = tvs[0] * c_a[i * C + c, pl.ds(0, 16)]
                for e in range(1, EA):
                    acc = acc + tvs[e] * c_a[i * C + c, pl.ds(e * 16, 16)]
                for e in range(EB):
                    acc = acc + tvs[EA + e] * c_b[i * C + c,
                                                  pl.ds(e * 16, 16)]
                sums.append(jnp.sum(acc))
            val = jnp.full((16,), sums[0], dtype=jnp.float32)
            for c in range(1, C):
                val = jnp.where(lanes == c, sums[c], val)
            idx = g * CV + i * C + lanes
            plsc.store_scatter(out_v, [idx], val, mask=lanes < C)
            return carry2

        lax.fori_loop(0, CB, b_body, 0, unroll=True)

    for r in range(NR - 1):
        fire(r, r)

    def outer(go, carry):
        for r in range(NR):
            g = go * NR + r
            gp = g + NR - 1

            @pl.when(gp < NCHUNK)
            def _():
                fire(gp, (r + NR - 1) % NR)

            for cp in descriptors(g, r):
                cp.wait()
            compute(g, r)
        return carry

    lax.fori_loop(0, NCHUNK // NR, outer, 0)

    pltpu.sync_copy(out_v, out_hbm.at[pl.ds(b0 * C, BPW * C)])


@functools.cache
def _w2v_call():
    return functools.partial(
        pl.kernel,
        out_type=jax.ShapeDtypeStruct((B * C,), jnp.float32),
        scratch_types=[
            pltpu.VMEM((BPW,), jnp.int32),
            pltpu.VMEM((BPW * C,), jnp.int32),
            pltpu.VMEM((NR, CB, 128), jnp.float32),
            pltpu.VMEM((NR, CB, 128), jnp.float32),
            pltpu.VMEM((NR, CV, 128), jnp.float32),
            pltpu.VMEM((NR, CV, 128), jnp.float32),
            pltpu.VMEM((BPW * C,), jnp.float32),
            [pltpu.SemaphoreType.DMA] * NR,
            [pltpu.SemaphoreType.DMA] * NR,
            [pltpu.SemaphoreType.DMA] * NR,
            [pltpu.SemaphoreType.DMA] * NR,
        ],
        mesh=plsc.VectorSubcoreMesh(core_axis_name="c", subcore_axis_name="s"),
        compiler_params=pltpu.CompilerParams(
            needs_layout_passes=False, use_tc_tiling_on_sc=False),
    )(_w2v_body)


def _pin(x):
    return jex_layout.with_layout_constraint(
        x, jex_layout.Layout(major_to_minor=(1, 0)))


def _split_panels(table):
    a = _pin(table[:, :128])
    b = _pin(jnp.pad(table[:, 128:], ((0, 0), (0, 64))))
    return a, b


@jax.jit
def kernel(target, context, target_table, context_table):
    tgt_idx = target.reshape(B).astype(jnp.int32)
    ctx_idx = context.reshape(B * C).astype(jnp.int32)
    ta, tb = _split_panels(target_table)
    ca, cb = _split_panels(context_table)
    out = _w2v_call()(tgt_idx, ctx_idx, ta, tb, ca, cb)
    return out.reshape(B, C)


# speedup vs baseline: 3.1607x; 1.0607x over previous
import functools

import jax
import jax.numpy as jnp
from jax import lax
from jax.experimental import layout as jex_layout
from jax.experimental import pallas as pl
from jax.experimental.pallas import tpu as pltpu
from jax.experimental.pallas import tpu_sc as plsc

VOCAB = 100000
EMBED = 192
B = 16384
C = 5

NC = 2
NS = 16
NW = NC * NS
BPW = B // NW
CB = 16
NCHUNK = BPW // CB
CV = CB * C
NR = 4
EA = 8
EB = 4


def _w2v_body(tgt_idx_hbm, ctx_idx_hbm, tgt_a_hbm, ctx_a_hbm, b_hbm, out_hbm,
              tgt_idx_v, ctx_idx_v, tgt_a_v, tgt_b_v, ctx_a_v, ctx_b_v,
              out_v, sem_ta, sem_tb, sem_ca, sem_cb):
    cid = lax.axis_index("c")
    sid = lax.axis_index("s")
    wid = sid * NC + cid
    b0 = wid * BPW

    pltpu.sync_copy(tgt_idx_hbm.at[pl.ds(b0, BPW)], tgt_idx_v)
    pltpu.sync_copy(ctx_idx_hbm.at[pl.ds(b0 * C, BPW * C)], ctx_idx_v)

    def descriptors(g, slot):
        ti = tgt_idx_v.at[pl.ds(g * CB, CB)]
        ci = ctx_idx_v.at[pl.ds(g * CV, CV)]
        return (
            pltpu.make_async_copy(
                tgt_a_hbm.at[ti], tgt_a_v.at[slot], sem_ta[slot]),
            pltpu.make_async_copy(
                b_hbm.at[ti], tgt_b_v.at[slot], sem_tb[slot]),
            pltpu.make_async_copy(
                ctx_a_hbm.at[ci], ctx_a_v.at[slot], sem_ca[slot]),
            pltpu.make_async_copy(
                b_hbm.at[ci], ctx_b_v.at[slot], sem_cb[slot]),
        )

    def fire(g, slot):
        for cp in descriptors(g, slot):
            cp.start()

    def compute(g, slot):
        lanes = lax.iota(jnp.int32, 16)
        t_a = tgt_a_v.at[slot]
        t_b = tgt_b_v.at[slot]
        c_a = ctx_a_v.at[slot]
        c_b = ctx_b_v.at[slot]

        def b_body(i, carry2):
            tvs = [t_a[i, pl.ds(e * 16, 16)] for e in range(EA)]
            tvs += [t_b[i, pl.ds(e * 16, 16)] for e in range(EB)]
            sums = []
            for c in range(C):
                acc = tvs[0] * c_a[i * C + c, pl.ds(0, 16)]
                for e in range(1, EA):
                    acc = acc + tvs[e] * c_a[i * C + c, pl.ds(e * 16, 16)]
                for e in range(EB):
                    acc = acc + tvs[EA + e] * c_b[i * C + c,
                                                  pl.ds(64 + e * 16, 16)]
                sums.append(jnp.sum(acc))
            val = jnp.full((16,), sums[0], dtype=jnp.float32)
            for c in range(1, C):
                val = jnp.where(lanes == c, sums[c], val)
            idx = g * CV + i * C + lanes
            plsc.store_scatter(out_v, [idx], val, mask=lanes < C)
            return carry2

        lax.fori_loop(0, CB, b_body, 0, unroll=True)

    for r in range(NR - 1):
        fire(r, r)

    def outer(go, carry):
        for r in range(NR):
            g = go * NR + r
            gp = g + NR - 1

            @pl.when(gp < NCHUNK)
            def _():
                fire(gp, (r + NR - 1) % NR)

            for cp in descriptors(g, r):
                cp.wait()
            compute(g, r)
        return carry

    lax.fori_loop(0, NCHUNK // NR, outer, 0)

    pltpu.sync_copy(out_v, out_hbm.at[pl.ds(b0 * C, BPW * C)])


@functools.cache
def _w2v_call():
    return functools.partial(
        pl.kernel,
        out_type=jax.ShapeDtypeStruct((B * C,), jnp.float32),
        scratch_types=[
            pltpu.VMEM((BPW,), jnp.int32),
            pltpu.VMEM((BPW * C,), jnp.int32),
            pltpu.VMEM((NR, CB, 128), jnp.float32),
            pltpu.VMEM((NR, CB, 128), jnp.float32),
            pltpu.VMEM((NR, CV, 128), jnp.float32),
            pltpu.VMEM((NR, CV, 128), jnp.float32),
            pltpu.VMEM((BPW * C,), jnp.float32),
            [pltpu.SemaphoreType.DMA] * NR,
            [pltpu.SemaphoreType.DMA] * NR,
            [pltpu.SemaphoreType.DMA] * NR,
            [pltpu.SemaphoreType.DMA] * NR,
        ],
        mesh=plsc.VectorSubcoreMesh(core_axis_name="c", subcore_axis_name="s"),
        compiler_params=pltpu.CompilerParams(
            needs_layout_passes=False, use_tc_tiling_on_sc=False),
    )(_w2v_body)


def _pin(x):
    return jex_layout.with_layout_constraint(
        x, jex_layout.Layout(major_to_minor=(1, 0)))


@jax.jit
def kernel(target, context, target_table, context_table):
    tgt_idx = target.reshape(B).astype(jnp.int32)
    ctx_idx = context.reshape(B * C).astype(jnp.int32)
    ta = _pin(target_table[:, :128])
    ca = _pin(context_table[:, :128])
    b = _pin(jnp.concatenate(
        [target_table[:, 128:], context_table[:, 128:]], axis=1))
    out = _w2v_call()(tgt_idx, ctx_idx, ta, ca, b)
    return out.reshape(B, C)
